# Initial kernel scaffold; baseline (speedup 1.0000x reference)
import jax
import jax.numpy as jnp
from jax.experimental import pallas as pl


def kernel(x, rbf, sbf, edge_idx_kj, edge_idx_ji, W_rbf1, W_rbf2, W_sbf1, W_sbf2, W_kj, b_kj, W_ji, b_ji, W_down, W_up, W_res1a, b_res1a, W_res1b, b_res1b, W_bs, b_bs, W_res2a, b_res2a, W_res2b, b_res2b, W_res3a, b_res3a, W_res3b, b_res3b):
    raise NotImplementedError("write your pallas kernel here")


# speedup vs baseline: 1.0911x; 1.0911x over previous
import functools

import jax
import jax.numpy as jnp
from jax.experimental import pallas as pl
from jax.experimental.pallas import tpu as pltpu

E = 320000
T = 1280000
D = 128
DDOWN = 64

BE = 2000
BT = 8000


def _swish(v):
    return v * jax.nn.sigmoid(v)


def _prologue_body(x_ref, rbf_ref, Wji_ref, bji_ref, Wkj_ref, bkj_ref,
                   Wrbf_ref, Wdown_ref, xji_ref, xkjd_ref):
    x = x_ref[...]
    xji = _swish(jnp.dot(x, Wji_ref[...], preferred_element_type=jnp.float32)
                 + bji_ref[...])
    xkj = _swish(jnp.dot(x, Wkj_ref[...], preferred_element_type=jnp.float32)
                 + bkj_ref[...])
    rbf_e = jnp.dot(rbf_ref[...], Wrbf_ref[...],
                    preferred_element_type=jnp.float32)
    xkj = xkj * rbf_e
    xkjd = _swish(jnp.dot(xkj, Wdown_ref[...],
                          preferred_element_type=jnp.float32))
    xji_ref[...] = xji
    xkjd_ref[...] = xkjd


def _prologue(x, rbf, W_ji, b_ji, W_kj, b_kj, W_rbf_c, W_down):
    grid = (E // BE,)
    return pl.pallas_call(
        _prologue_body,
        grid=grid,
        in_specs=[
            pl.BlockSpec((BE, D), lambda i: (i, 0)),
            pl.BlockSpec((BE, rbf.shape[1]), lambda i: (i, 0)),
            pl.BlockSpec((D, D), lambda i: (0, 0)),
            pl.BlockSpec((D,), lambda i: (0,)),
            pl.BlockSpec((D, D), lambda i: (0, 0)),
            pl.BlockSpec((D,), lambda i: (0,)),
            pl.BlockSpec((rbf.shape[1], D), lambda i: (0, 0)),
            pl.BlockSpec((D, DDOWN), lambda i: (0, 0)),
        ],
        out_specs=[
            pl.BlockSpec((BE, D), lambda i: (i, 0)),
            pl.BlockSpec((BE, DDOWN), lambda i: (i, 0)),
        ],
        out_shape=[
            jax.ShapeDtypeStruct((E, D), jnp.float32),
            jax.ShapeDtypeStruct((E, DDOWN), jnp.float32),
        ],
    )(x, rbf, W_ji, b_ji, W_kj, b_kj, W_rbf_c, W_down)


def _sbf_body(sbf_ref, Wsbf_ref, out_ref):
    out_ref[...] = jnp.dot(sbf_ref[...], Wsbf_ref[...],
                           preferred_element_type=jnp.float32)


def _sbf_embed(sbf, W_sbf_c):
    grid = (T // BT,)
    return pl.pallas_call(
        _sbf_body,
        grid=grid,
        in_specs=[
            pl.BlockSpec((BT, sbf.shape[1]), lambda i: (i, 0)),
            pl.BlockSpec((sbf.shape[1], DDOWN), lambda i: (0, 0)),
        ],
        out_specs=pl.BlockSpec((BT, DDOWN), lambda i: (i, 0)),
        out_shape=jax.ShapeDtypeStruct((T, DDOWN), jnp.float32),
    )(sbf, W_sbf_c)


def _epilogue_body(seg_ref, xji_ref, x_ref, Wup_ref,
                   Wr1a_ref, br1a_ref, Wr1b_ref, br1b_ref,
                   Wbs_ref, bbs_ref,
                   Wr2a_ref, br2a_ref, Wr2b_ref, br2b_ref,
                   Wr3a_ref, br3a_ref, Wr3b_ref, br3b_ref,
                   out_ref):
    def mm(a, b):
        return jnp.dot(a, b, preferred_element_type=jnp.float32)

    def res(h, Wa, ba, Wb, bb):
        return h + _swish(mm(_swish(mm(h, Wa) + ba), Wb) + bb)

    xkj = _swish(mm(seg_ref[...], Wup_ref[...]))
    h = xji_ref[...] + xkj
    h = res(h, Wr1a_ref[...], br1a_ref[...], Wr1b_ref[...], br1b_ref[...])
    h = _swish(mm(h, Wbs_ref[...]) + bbs_ref[...])
    out = h + x_ref[...]
    out = res(out, Wr2a_ref[...], br2a_ref[...], Wr2b_ref[...], br2b_ref[...])
    out = res(out, Wr3a_ref[...], br3a_ref[...], Wr3b_ref[...], br3b_ref[...])
    out_ref[...] = out


def _epilogue(seg, xji, x, W_up,
              W_res1a, b_res1a, W_res1b, b_res1b, W_bs, b_bs,
              W_res2a, b_res2a, W_res2b, b_res2b,
              W_res3a, b_res3a, W_res3b, b_res3b):
    grid = (E // BE,)
    full = lambda shape: pl.BlockSpec(shape, lambda i: tuple(0 for _ in shape))
    return pl.pallas_call(
        _epilogue_body,
        grid=grid,
        in_specs=[
            pl.BlockSpec((BE, DDOWN), lambda i: (i, 0)),
            pl.BlockSpec((BE, D), lambda i: (i, 0)),
            pl.BlockSpec((BE, D), lambda i: (i, 0)),
            full((DDOWN, D)),
            full((D, D)), full((D,)), full((D, D)), full((D,)),
            full((D, D)), full((D,)),
            full((D, D)), full((D,)), full((D, D)), full((D,)),
            full((D, D)), full((D,)), full((D, D)), full((D,)),
        ],
        out_specs=pl.BlockSpec((BE, D), lambda i: (i, 0)),
        out_shape=jax.ShapeDtypeStruct((E, D), jnp.float32),
    )(seg, xji, x, W_up,
      W_res1a, b_res1a, W_res1b, b_res1b, W_bs, b_bs,
      W_res2a, b_res2a, W_res2b, b_res2b,
      W_res3a, b_res3a, W_res3b, b_res3b)


def kernel(x, rbf, sbf, edge_idx_kj, edge_idx_ji, W_rbf1, W_rbf2, W_sbf1,
           W_sbf2, W_kj, b_kj, W_ji, b_ji, W_down, W_up, W_res1a, b_res1a,
           W_res1b, b_res1b, W_bs, b_bs, W_res2a, b_res2a, W_res2b, b_res2b,
           W_res3a, b_res3a, W_res3b, b_res3b):
    W_rbf_c = W_rbf1 @ W_rbf2
    W_sbf_c = W_sbf1 @ W_sbf2

    xji, xkjd = _prologue(x, rbf, W_ji, b_ji, W_kj, b_kj, W_rbf_c, W_down)
    sbf_e = _sbf_embed(sbf, W_sbf_c)

    msg = jnp.take(xkjd, edge_idx_kj, axis=0) * sbf_e
    seg = jax.ops.segment_sum(msg, edge_idx_ji, num_segments=E)

    return _epilogue(seg, xji, x, W_up,
                     W_res1a, b_res1a, W_res1b, b_res1b, W_bs, b_bs,
                     W_res2a, b_res2a, W_res2b, b_res2b,
                     W_res3a, b_res3a, W_res3b, b_res3b)


# speedup vs baseline: 1.7777x; 1.6293x over previous
import functools

import jax
import jax.numpy as jnp
from jax import lax
from jax.experimental import pallas as pl
from jax.experimental.pallas import tpu as pltpu
from jax.experimental.pallas import tpu_sc as plsc

E = 320000
T = 1280000
D = 128
DDOWN = 64

BE = 2000
BT = 8000

SC_NC = 2
SC_NS = 16
NW = SC_NC * SC_NS
TW = T // NW
NVEC = TW // 16
DW = 2 * DDOWN
NPART = 100
EB = E // NPART
RPT = EB // SC_NS
CH = 96
TCAP = TW + CH
ZR = 40
JB = 2000
LRB = 14
LRM = (1 << LRB) - 1

_GDN = lax.GatherDimensionNumbers(
    offset_dims=(), collapsed_slice_dims=(0,), start_index_map=(0,))


def _lane_perm(x, idx):
    return lax.gather(x, idx[:, None], _GDN, slice_sizes=(1,),
                      mode=lax.GatherScatterMode.PROMISE_IN_BOUNDS)


def _swish(v):
    return v * jax.nn.sigmoid(v)


def _prologue_body(x_ref, rbf_ref, Wji_ref, bji_ref, Wkj_ref, bkj_ref,
                   Wrbf_ref, Wdown_ref, xji_ref, xkjd_ref):
    x = x_ref[...]
    xji = _swish(jnp.dot(x, Wji_ref[...], preferred_element_type=jnp.float32)
                 + bji_ref[...])
    xkj = _swish(jnp.dot(x, Wkj_ref[...], preferred_element_type=jnp.float32)
                 + bkj_ref[...])
    rbf_e = jnp.dot(rbf_ref[...], Wrbf_ref[...],
                    preferred_element_type=jnp.float32)
    xkj = xkj * rbf_e
    xkjd = _swish(jnp.dot(xkj, Wdown_ref[...],
                          preferred_element_type=jnp.float32))
    xji_ref[...] = xji
    xkjd_ref[...] = jnp.concatenate([xkjd, xkjd], axis=1)


def _prologue(x, rbf, W_ji, b_ji, W_kj, b_kj, W_rbf_c, W_down):
    grid = (E // BE,)
    return pl.pallas_call(
        _prologue_body,
        grid=grid,
        in_specs=[
            pl.BlockSpec((BE, D), lambda i: (i, 0)),
            pl.BlockSpec((BE, rbf.shape[1]), lambda i: (i, 0)),
            pl.BlockSpec((D, D), lambda i: (0, 0)),
            pl.BlockSpec((D,), lambda i: (0,)),
            pl.BlockSpec((D, D), lambda i: (0, 0)),
            pl.BlockSpec((D,), lambda i: (0,)),
            pl.BlockSpec((rbf.shape[1], D), lambda i: (0, 0)),
            pl.BlockSpec((D, DDOWN), lambda i: (0, 0)),
        ],
        out_specs=[
            pl.BlockSpec((BE, D), lambda i: (i, 0)),
            pl.BlockSpec((BE, 2 * DDOWN), lambda i: (i, 0)),
        ],
        out_shape=[
            jax.ShapeDtypeStruct((E, D), jnp.float32),
            jax.ShapeDtypeStruct((E, 2 * DDOWN), jnp.float32),
        ],
    )(x, rbf, W_ji, b_ji, W_kj, b_kj, W_rbf_c, W_down)


def _sbf_body(sbf_ref, Wsbf_ref, out_ref):
    e = jnp.dot(sbf_ref[...], Wsbf_ref[...],
                preferred_element_type=jnp.float32)
    out_ref[...] = jnp.concatenate([e, e], axis=1)


def _sbf_embed(sbf, W_sbf_c):
    grid = (T // BT,)
    return pl.pallas_call(
        _sbf_body,
        grid=grid,
        in_specs=[
            pl.BlockSpec((BT, sbf.shape[1]), lambda i: (i, 0)),
            pl.BlockSpec((sbf.shape[1], DDOWN), lambda i: (0, 0)),
        ],
        out_specs=pl.BlockSpec((BT, 2 * DDOWN), lambda i: (i, 0)),
        out_shape=jax.ShapeDtypeStruct((T, 2 * DDOWN), jnp.float32),
    )(sbf, W_sbf_c)


def _sc_middle_body(xkjd_hbm, sbf_hbm, kj_hbm, ji_hbm, out_hbm,
                    binbuf, jibuf, histf, offf, gtid, kjc, lrow,
                    xbuf, sbuf, zbuf, pstart, seg_sp,
                    semj, semk, semx, sems):
    c = lax.axis_index("c")
    s = lax.axis_index("s")
    wid = s * SC_NC + c
    t0 = wid * TW
    lane = jnp.arange(16, dtype=jnp.int32)
    zf = jnp.zeros((16,), jnp.float32)
    zi = jnp.zeros((16,), jnp.int32)
    ones = jnp.ones((16,), jnp.int32)

    def zrow(i, _):
        for q in range(DDOWN // 16):
            zbuf[i, pl.ds(q * 16, 16)] = zf
        return 0
    lax.fori_loop(0, ZR, zrow, 0)

    for v in range((TCAP - TW) // 16):
        binbuf[pl.ds(TW + v * 16, 16)] = zi

    def zhist(i, _):
        histf[pl.ds(i * 16, 16)] = zi
        return 0
    lax.fori_loop(0, NPART, zhist, 0)

    def hist_blk(blk, _):
        pltpu.async_copy(ji_hbm.at[pl.ds(t0 + blk * JB, JB)], jibuf, semj
                         ).wait()

        def hv(v, _):
            jiv = jibuf[pl.ds(v * 16, 16)]
            p = jiv // EB
            plsc.addupdate_scatter(histf, [p * 16 + lane], ones)
            return 0
        lax.fori_loop(0, JB // 16, hv, 0)
        return 0
    lax.fori_loop(0, TW // JB, hist_blk, 0)

    def off_p(p, carry):
        h = histf[pl.ds(p * 16, 16)]
        incl = h
        for d in (1, 2, 4, 8):
            g = _lane_perm(incl, jnp.maximum(lane - d, 0))
            incl = incl + jnp.where(lane >= d, g, 0)
        offf[pl.ds(p * 16, 16)] = incl - h + carry
        pstart[p] = carry
        return carry + incl[15]
    total = lax.fori_loop(0, NPART, off_p, jnp.int32(0))
    pstart[NPART] = total

    def perm_blk(blk, _):
        pltpu.async_copy(ji_hbm.at[pl.ds(t0 + blk * JB, JB)], jibuf, semj
                         ).wait()

        def pv(v, _):
            jiv = jibuf[pl.ds(v * 16, 16)]
            p = jiv // EB
            idx = p * 16 + lane
            base = plsc.load_gather(offf, [idx])
            plsc.store_scatter(offf, [idx], base + 1)
            tid = blk * JB + v * 16 + lane
            enc = (tid << LRB) | (jiv - p * EB)
            plsc.store_scatter(binbuf, [base], enc)
            return 0
        lax.fori_loop(0, JB // 16, pv, 0)
        return 0
    lax.fori_loop(0, TW // JB, perm_blk, 0)

    def pass_body(p, _):
        for k in range(RPT // ZR):
            pltpu.sync_copy(zbuf, seg_sp.at[pl.ds(s * RPT + k * ZR, ZR)])
        plsc.subcore_barrier()

        start = pstart[p]
        cnt = pstart[p + 1] - start
        nch = (cnt + CH - 1) // CH

        def stage1(ch):
            b4 = jnp.bitwise_and(ch, 3)
            base = start + ch * CH
            for v in range(CH // 16):
                enc = binbuf[pl.ds(base + v * 16, 16)]
                valid = (ch * CH + v * 16 + lane) < cnt
                gtid[b4, pl.ds(v * 16, 16)] = (enc >> LRB) + t0
                lrow[b4, pl.ds(v * 16, 16)] = jnp.where(
                    valid, enc & LRM, EB + s)
            pltpu.async_copy(kj_hbm.at[gtid.at[b4]], kjc.at[b4], semk.at[b4])

        def stage2(ch):
            b4 = jnp.bitwise_and(ch, 3)
            b2 = jnp.bitwise_and(ch, 1)
            pltpu.make_async_copy(kj_hbm.at[gtid.at[b4]], kjc.at[b4],
                                  semk.at[b4]).wait()
            pltpu.async_copy(xkjd_hbm.at[kjc.at[b4]], xbuf.at[b2],
                             semx.at[b2])
            pltpu.async_copy(sbf_hbm.at[gtid.at[b4]], sbuf.at[b2],
                             sems.at[b2])

        @pl.when(nch > 0)
        def _():
            stage1(jnp.int32(0))
            stage2(jnp.int32(0))

        @pl.when(nch > 1)
        def _():
            stage1(jnp.int32(1))

        def chunk(ch, _):
            b4 = jnp.bitwise_and(ch, 3)
            b2 = jnp.bitwise_and(ch, 1)
            pltpu.make_async_copy(xkjd_hbm.at[kjc.at[b4]], xbuf.at[b2],
                                  semx.at[b2]).wait()
            pltpu.make_async_copy(sbf_hbm.at[gtid.at[b4]], sbuf.at[b2],
                                  sems.at[b2]).wait()

            @pl.when(ch + 2 < nch)
            def _():
                stage1(ch + 2)

            def mul(r, _):
                for q in range(DDOWN // 16):
                    xbuf[b2, r, pl.ds(q * 16, 16)] = (
                        xbuf[b2, r, pl.ds(q * 16, 16)]
                        * sbuf[b2, r, pl.ds(q * 16, 16)])
                return 0
            lax.fori_loop(0, CH, mul, 0)

            @pl.when(ch + 1 < nch)
            def _():
                stage2(ch + 1)

            pltpu.sync_copy(xbuf.at[b2], seg_sp.at[lrow.at[b4]], add=True)
            return 0
        lax.fori_loop(0, nch, chunk, 0)
        plsc.subcore_barrier()

        pltpu.sync_copy(seg_sp.at[pl.ds(s * RPT, RPT)],
                        out_hbm.at[c, pl.ds(p * EB + s * RPT, RPT)])
        return 0
    lax.fori_loop(0, NPART, pass_body, 0)


def _sc_middle(xkjd, sbf_e, edge_idx_kj, edge_idx_ji):
    mesh = plsc.VectorSubcoreMesh(core_axis_name="c", subcore_axis_name="s")
    f = pl.kernel(
        _sc_middle_body,
        mesh=mesh,
        compiler_params=pltpu.CompilerParams(needs_layout_passes=False,
                                             use_tc_tiling_on_sc=False),
        out_type=jax.ShapeDtypeStruct((SC_NC, E, DW), jnp.float32),
        scratch_types=[
            pltpu.VMEM((TCAP,), jnp.int32),
            pltpu.VMEM((JB,), jnp.int32),
            pltpu.VMEM((NPART * 16,), jnp.int32),
            pltpu.VMEM((NPART * 16,), jnp.int32),
            pltpu.VMEM((4, CH), jnp.int32),
            pltpu.VMEM((4, CH), jnp.int32),
            pltpu.VMEM((4, CH), jnp.int32),
            pltpu.VMEM((2, CH, DW), jnp.float32),
            pltpu.VMEM((2, CH, DW), jnp.float32),
            pltpu.VMEM((ZR, DW), jnp.float32),
            pltpu.SMEM((NPART + 1,), jnp.int32),
            pltpu.VMEM_SHARED((EB + SC_NS, DW), jnp.float32),
            pltpu.SemaphoreType.DMA,
            pltpu.SemaphoreType.DMA((4,)),
            pltpu.SemaphoreType.DMA((2,)),
            pltpu.SemaphoreType.DMA((2,)),
        ],
    )
    return f(xkjd, sbf_e, edge_idx_kj, edge_idx_ji)


def _epilogue_body(seg0_ref, seg1_ref, xji_ref, x_ref, Wup_ref,
                   Wr1a_ref, br1a_ref, Wr1b_ref, br1b_ref,
                   Wbs_ref, bbs_ref,
                   Wr2a_ref, br2a_ref, Wr2b_ref, br2b_ref,
                   Wr3a_ref, br3a_ref, Wr3b_ref, br3b_ref,
                   out_ref):
    def mm(a, b):
        return jnp.dot(a, b, preferred_element_type=jnp.float32)

    def res(h, Wa, ba, Wb, bb):
        return h + _swish(mm(_swish(mm(h, Wa) + ba), Wb) + bb)

    xkj = _swish(mm(seg0_ref[...] + seg1_ref[...], Wup_ref[...]))
    h = xji_ref[...] + xkj
    h = res(h, Wr1a_ref[...], br1a_ref[...], Wr1b_ref[...], br1b_ref[...])
    h = _swish(mm(h, Wbs_ref[...]) + bbs_ref[...])
    out = h + x_ref[...]
    out = res(out, Wr2a_ref[...], br2a_ref[...], Wr2b_ref[...], br2b_ref[...])
    out = res(out, Wr3a_ref[...], br3a_ref[...], Wr3b_ref[...], br3b_ref[...])
    out_ref[...] = out


def _epilogue(seg0, seg1, xji, x, W_up,
              W_res1a, b_res1a, W_res1b, b_res1b, W_bs, b_bs,
              W_res2a, b_res2a, W_res2b, b_res2b,
              W_res3a, b_res3a, W_res3b, b_res3b):
    grid = (E // BE,)
    full = lambda shape: pl.BlockSpec(shape, lambda i: tuple(0 for _ in shape))
    return pl.pallas_call(
        _epilogue_body,
        grid=grid,
        in_specs=[
            pl.BlockSpec((BE, DW), lambda i: (i, 0)),
            pl.BlockSpec((BE, DW), lambda i: (i, 0)),
            pl.BlockSpec((BE, D), lambda i: (i, 0)),
            pl.BlockSpec((BE, D), lambda i: (i, 0)),
            full((DW, D)),
            full((D, D)), full((D,)), full((D, D)), full((D,)),
            full((D, D)), full((D,)),
            full((D, D)), full((D,)), full((D, D)), full((D,)),
            full((D, D)), full((D,)), full((D, D)), full((D,)),
        ],
        out_specs=pl.BlockSpec((BE, D), lambda i: (i, 0)),
        out_shape=jax.ShapeDtypeStruct((E, D), jnp.float32),
    )(seg0, seg1, xji, x, W_up,
      W_res1a, b_res1a, W_res1b, b_res1b, W_bs, b_bs,
      W_res2a, b_res2a, W_res2b, b_res2b,
      W_res3a, b_res3a, W_res3b, b_res3b)


def kernel(x, rbf, sbf, edge_idx_kj, edge_idx_ji, W_rbf1, W_rbf2, W_sbf1,
           W_sbf2, W_kj, b_kj, W_ji, b_ji, W_down, W_up, W_res1a, b_res1a,
           W_res1b, b_res1b, W_bs, b_bs, W_res2a, b_res2a, W_res2b, b_res2b,
           W_res3a, b_res3a, W_res3b, b_res3b):
    W_rbf_c = W_rbf1 @ W_rbf2
    W_sbf_c = W_sbf1 @ W_sbf2
    W_up_p = jnp.concatenate([W_up, jnp.zeros_like(W_up)], axis=0)

    xji, xkjd = _prologue(x, rbf, W_ji, b_ji, W_kj, b_kj, W_rbf_c, W_down)
    sbf_e = _sbf_embed(sbf, W_sbf_c)

    partials = _sc_middle(xkjd, sbf_e,
                          edge_idx_kj.astype(jnp.int32),
                          edge_idx_ji.astype(jnp.int32))

    return _epilogue(partials[0], partials[1], xji, x, W_up_p,
                     W_res1a, b_res1a, W_res1b, b_res1b, W_bs, b_bs,
                     W_res2a, b_res2a, W_res2b, b_res2b,
                     W_res3a, b_res3a, W_res3b, b_res3b)
